# baseline (speedup 1.0000x reference)
import jax
import jax.numpy as jnp
from jax import lax
from jax.experimental import pallas as pl
from jax.experimental.pallas import tpu as pltpu
from jax.experimental.pallas import tpu_sc as plsc

_L = 16
_NW = 32


def _dots(wu_v, wv_v, urows_v, vrows_v, tmp_v, s_loc):
    iota = lax.iota(jnp.int32, _L)
    nchunk = urows_v.shape[1] // _L
    npairs = urows_v.shape[0]

    def group_body(g, carry):
        g0 = g * _L
        for p in range(_L):
            r = g0 + p
            acc = urows_v[r, pl.ds(0, _L)] * vrows_v[r, pl.ds(0, _L)]
            for c in range(1, nchunk):
                acc = acc + (urows_v[r, pl.ds(c * _L, _L)]
                             * vrows_v[r, pl.ds(c * _L, _L)])
            tmp_v[p, pl.ds(0, _L)] = acc
        s16 = plsc.load_gather(tmp_v, [iota, jnp.zeros((_L,), jnp.int32)])
        for col in range(1, _L):
            s16 = s16 + plsc.load_gather(
                tmp_v, [iota, jnp.full((_L,), col, jnp.int32)])
        s_loc[pl.ds(g0, _L)] = s16
        return carry

    lax.fori_loop(0, npairs // _L, group_body, 0)


def _sc_stage_a(wu_hbm, wv_hbm, ie_hbm, ib_hbm, oe_hbm, ob_hbm,
                s_hbm, b_hbm,
                wu_v, wv_v, urows_v, vrows_v, ib_v, ob_v, tmp_v,
                s_loc, b_loc, wub_v, wvb_v, sem_u, sem_v, sem_b):
    wid = lax.axis_index("s") * 2 + lax.axis_index("c")
    spw = wu_v.shape[0]
    bpw = wub_v.shape[0]
    sbase = wid * spw
    bbase = wid * bpw

    pltpu.sync_copy(wu_hbm.at[pl.ds(sbase, spw)], wu_v)
    pltpu.sync_copy(wv_hbm.at[pl.ds(sbase, spw)], wv_v)
    cp_ib = pltpu.async_copy(ib_hbm, ib_v, sem_b)
    cp_ob = pltpu.async_copy(ob_hbm, ob_v, sem_b)
    cp_u = pltpu.async_copy(ie_hbm.at[wu_v], urows_v, sem_u)
    cp_v = pltpu.async_copy(oe_hbm.at[wv_v], vrows_v, sem_v)
    pltpu.sync_copy(wu_hbm.at[pl.ds(bbase, bpw)], wub_v)
    pltpu.sync_copy(wv_hbm.at[pl.ds(bbase, bpw)], wvb_v)
    cp_ib.wait()
    cp_ob.wait()

    def bias_body(g, carry):
        g0 = g * _L
        u16 = wub_v[pl.ds(g0, _L)]
        v16 = wvb_v[pl.ds(g0, _L)]
        b_loc[pl.ds(g0, _L)] = (plsc.load_gather(ib_v, [u16])
                                + plsc.load_gather(ob_v, [v16]))
        return carry

    lax.fori_loop(0, bpw // _L, bias_body, 0)
    cp_u.wait()
    cp_v.wait()
    _dots(wu_v, wv_v, urows_v, vrows_v, tmp_v, s_loc)

    pltpu.sync_copy(s_loc, s_hbm.at[pl.ds(sbase, spw)])
    pltpu.sync_copy(b_loc, b_hbm.at[pl.ds(bbase, bpw)])


def _sc_stage_b(wu_hbm, wv_hbm, ie_hbm, oe_hbm,
                s_hbm,
                wu_v, wv_v, urows_v, vrows_v, tmp_v, s_loc,
                sem_u, sem_v):
    wid = lax.axis_index("s") * 2 + lax.axis_index("c")
    spw = wu_v.shape[0]
    half = spw * _NW
    sbase = half + wid * spw

    pltpu.sync_copy(wu_hbm.at[pl.ds(sbase, spw)], wu_v)
    pltpu.sync_copy(wv_hbm.at[pl.ds(sbase, spw)], wv_v)
    cp_u = pltpu.async_copy(ie_hbm.at[wu_v], urows_v, sem_u)
    cp_v = pltpu.async_copy(oe_hbm.at[wv_v], vrows_v, sem_v)
    cp_u.wait()
    cp_v.wait()
    _dots(wu_v, wv_v, urows_v, vrows_v, tmp_v, s_loc)

    pltpu.sync_copy(s_loc, s_hbm.at[pl.ds(wid * spw, spw)])


_SC_PARAMS = pltpu.CompilerParams(
    needs_layout_passes=False,
    disable_bounds_checks=True,
    disable_semaphore_checks=True,
)


def _make_sc_a(B, V, D):
    spw = (B // 2) // _NW
    bpw = B // _NW
    mesh = plsc.VectorSubcoreMesh(core_axis_name="c", subcore_axis_name="s")
    return pl.kernel(
        _sc_stage_a,
        out_type=(jax.ShapeDtypeStruct((B // 2,), jnp.float32),
                  jax.ShapeDtypeStruct((B,), jnp.float32)),
        mesh=mesh,
        compiler_params=_SC_PARAMS,
        scratch_types=[
            pltpu.VMEM((spw,), jnp.int32),
            pltpu.VMEM((spw,), jnp.int32),
            pltpu.VMEM((spw, D), jnp.float32),
            pltpu.VMEM((spw, D), jnp.float32),
            pltpu.VMEM((V,), jnp.float32),
            pltpu.VMEM((V,), jnp.float32),
            pltpu.VMEM((_L, _L), jnp.float32),
            pltpu.VMEM((spw,), jnp.float32),
            pltpu.VMEM((bpw,), jnp.float32),
            pltpu.VMEM((bpw,), jnp.int32),
            pltpu.VMEM((bpw,), jnp.int32),
            pltpu.SemaphoreType.DMA,
            pltpu.SemaphoreType.DMA,
            pltpu.SemaphoreType.DMA,
        ],
    )


def _make_sc_b(B, V, D):
    spw = (B // 2) // _NW
    mesh = plsc.VectorSubcoreMesh(core_axis_name="c", subcore_axis_name="s")
    return pl.kernel(
        _sc_stage_b,
        out_type=jax.ShapeDtypeStruct((B // 2,), jnp.float32),
        mesh=mesh,
        compiler_params=_SC_PARAMS,
        scratch_types=[
            pltpu.VMEM((spw,), jnp.int32),
            pltpu.VMEM((spw,), jnp.int32),
            pltpu.VMEM((spw, D), jnp.float32),
            pltpu.VMEM((spw, D), jnp.float32),
            pltpu.VMEM((_L, _L), jnp.float32),
            pltpu.VMEM((spw,), jnp.float32),
            pltpu.SemaphoreType.DMA,
            pltpu.SemaphoreType.DMA,
        ],
    )


def _bcast_body(s_ref, b_ref, o_ref, srow_ref):
    i = pl.program_id(0)
    TI, BH = o_ref.shape

    @pl.when(i == 0)
    def _():
        srow_ref[...] = s_ref[...].reshape(1, BH)

    ident = (lax.broadcasted_iota(jnp.int32, (TI, TI), 0)
             == lax.broadcasted_iota(jnp.int32, (TI, TI), 1)
             ).astype(jnp.float32)
    brow = b_ref[pl.ds(i * TI, TI)].reshape(1, TI)
    bcol = lax.dot_general(ident, brow, (((1,), (1,)), ((), ())),
                           preferred_element_type=jnp.float32)
    o_ref[...] = bcol + srow_ref[...]


def _bcast_body2(s_ref, b_ref, o_prev_ref, o_ref, srow_ref):
    del o_prev_ref
    _bcast_body(s_ref, b_ref, o_ref, srow_ref)


def kernel(word_u, word_v, in_embed_w, in_bias_w, out_embed_w, out_bias_w):
    B = word_u.shape[0]
    V, D = in_embed_w.shape
    BH = B // 2
    TI = 256
    wu = word_u.astype(jnp.int32)
    wv = word_v.astype(jnp.int32)

    s_a, b = _make_sc_a(B, V, D)(
        wu, wv, in_embed_w, in_bias_w.reshape(V),
        out_embed_w, out_bias_w.reshape(V))
    s_b = _make_sc_b(B, V, D)(wu, wv, in_embed_w, out_embed_w)

    o1 = pl.pallas_call(
        _bcast_body,
        grid=(B // TI,),
        in_specs=[
            pl.BlockSpec((BH,), lambda i: (0,)),
            pl.BlockSpec((B,), lambda i: (0,)),
        ],
        out_specs=pl.BlockSpec((TI, BH), lambda i: (i, 0)),
        out_shape=jax.ShapeDtypeStruct((B, B), jnp.float32),
        scratch_shapes=[pltpu.VMEM((1, BH), jnp.float32)],
    )(s_a, b)

    return pl.pallas_call(
        _bcast_body2,
        grid=(B // TI,),
        in_specs=[
            pl.BlockSpec((BH,), lambda i: (0,)),
            pl.BlockSpec((B,), lambda i: (0,)),
            pl.BlockSpec(memory_space=pl.ANY),
        ],
        out_specs=pl.BlockSpec((TI, BH), lambda i: (i, 1)),
        out_shape=jax.ShapeDtypeStruct((B, B), jnp.float32),
        scratch_shapes=[pltpu.VMEM((1, BH), jnp.float32)],
        input_output_aliases={2: 0},
    )(s_b, b, o1)


# speedup vs baseline: 1.0631x; 1.0631x over previous
import jax
import jax.numpy as jnp
from jax import lax
from jax.experimental import pallas as pl
from jax.experimental.pallas import tpu as pltpu
from jax.experimental.pallas import tpu_sc as plsc

_L = 16


def _sc_stage(wu_hbm, wv_hbm, ie_hbm, ib_hbm, oe_hbm, ob_hbm,
              s_hbm, b_hbm,
              wu_v, wv_v, urows_v, vrows_v, ib_v, ob_v, tmp_v,
              s_loc, b_loc, sem_u, sem_v, sem_b):
    nc = 2
    pw = wu_v.shape[0]
    wid = lax.axis_index("s") * nc + lax.axis_index("c")
    base = wid * pw

    pltpu.sync_copy(wu_hbm.at[pl.ds(base, pw)], wu_v)
    pltpu.sync_copy(wv_hbm.at[pl.ds(base, pw)], wv_v)
    cp_ib = pltpu.async_copy(ib_hbm, ib_v, sem_b)
    cp_ob = pltpu.async_copy(ob_hbm, ob_v, sem_b)
    ph = pw // 2
    cp_u1 = pltpu.async_copy(ie_hbm.at[wu_v.at[pl.ds(0, ph)]],
                             urows_v.at[pl.ds(0, ph), :], sem_u)
    cp_v1 = pltpu.async_copy(oe_hbm.at[wv_v.at[pl.ds(0, ph)]],
                             vrows_v.at[pl.ds(0, ph), :], sem_v)
    cp_u2 = pltpu.async_copy(ie_hbm.at[wu_v.at[pl.ds(ph, ph)]],
                             urows_v.at[pl.ds(ph, ph), :], sem_u)
    cp_v2 = pltpu.async_copy(oe_hbm.at[wv_v.at[pl.ds(ph, ph)]],
                             vrows_v.at[pl.ds(ph, ph), :], sem_v)

    iota = lax.iota(jnp.int32, _L)
    d = urows_v.shape[1]
    nchunk = d // _L

    cp_ib.wait()
    cp_ob.wait()

    def bias_body(g, carry):
        g0 = g * _L
        wu16 = wu_v[pl.ds(g0, _L)]
        wv16 = wv_v[pl.ds(g0, _L)]
        b_loc[pl.ds(g0, _L)] = (plsc.load_gather(ib_v, [wu16])
                                + plsc.load_gather(ob_v, [wv16]))
        return carry

    lax.fori_loop(0, pw // _L, bias_body, 0)

    def group_body(g, carry):
        g0 = g * _L
        for p in range(_L):
            r = g0 + p
            acc = urows_v[r, pl.ds(0, _L)] * vrows_v[r, pl.ds(0, _L)]
            for c in range(1, nchunk):
                acc = acc + (urows_v[r, pl.ds(c * _L, _L)]
                             * vrows_v[r, pl.ds(c * _L, _L)])
            tmp_v[p, pl.ds(0, _L)] = acc
        s16 = plsc.load_gather(tmp_v, [iota, jnp.zeros((_L,), jnp.int32)])
        for col in range(1, _L):
            s16 = s16 + plsc.load_gather(
                tmp_v, [iota, jnp.full((_L,), col, jnp.int32)])
        s_loc[pl.ds(g0, _L)] = s16
        return carry

    cp_u1.wait()
    cp_v1.wait()
    lax.fori_loop(0, ph // _L, group_body, 0)
    cp_u2.wait()
    cp_v2.wait()
    lax.fori_loop(ph // _L, pw // _L, group_body, 0)

    pltpu.sync_copy(s_loc, s_hbm.at[pl.ds(base, pw)])
    pltpu.sync_copy(b_loc, b_hbm.at[pl.ds(base, pw)])


def _make_sc_stage(B, V, D):
    nw = 32
    pw = B // nw
    mesh = plsc.VectorSubcoreMesh(core_axis_name="c", subcore_axis_name="s")
    return pl.kernel(
        _sc_stage,
        out_type=(jax.ShapeDtypeStruct((B,), jnp.float32),
                  jax.ShapeDtypeStruct((B,), jnp.float32)),
        mesh=mesh,
        compiler_params=pltpu.CompilerParams(
            needs_layout_passes=False,
            disable_bounds_checks=True,
            disable_semaphore_checks=True,
        ),
        scratch_types=[
            pltpu.VMEM((pw,), jnp.int32),
            pltpu.VMEM((pw,), jnp.int32),
            pltpu.VMEM((pw, D), jnp.float32),
            pltpu.VMEM((pw, D), jnp.float32),
            pltpu.VMEM((V,), jnp.float32),
            pltpu.VMEM((V,), jnp.float32),
            pltpu.VMEM((_L, _L), jnp.float32),
            pltpu.VMEM((pw,), jnp.float32),
            pltpu.VMEM((pw,), jnp.float32),
            pltpu.SemaphoreType.DMA,
            pltpu.SemaphoreType.DMA,
            pltpu.SemaphoreType.DMA,
        ],
    )


def _bcast_kernel(s_ref, b_ref, o_ref, srow_ref):
    i = pl.program_id(0)
    TI, B = o_ref.shape

    @pl.when(i == 0)
    def _():
        srow_ref[...] = s_ref[...].reshape(1, B)

    ident = (lax.broadcasted_iota(jnp.int32, (TI, TI), 0)
             == lax.broadcasted_iota(jnp.int32, (TI, TI), 1)
             ).astype(jnp.float32)
    brow = b_ref[pl.ds(i * TI, TI)].reshape(1, TI)
    bcol = lax.dot_general(ident, brow, (((1,), (1,)), ((), ())),
                           preferred_element_type=jnp.float32)
    o_ref[...] = bcol + srow_ref[...]


def kernel(word_u, word_v, in_embed_w, in_bias_w, out_embed_w, out_bias_w):
    B = word_u.shape[0]
    V, D = in_embed_w.shape
    wu = word_u.astype(jnp.int32)
    wv = word_v.astype(jnp.int32)

    s, b = _make_sc_stage(B, V, D)(
        wu, wv, in_embed_w, in_bias_w.reshape(V),
        out_embed_w, out_bias_w.reshape(V))

    TI = 256
    return pl.pallas_call(
        _bcast_kernel,
        grid=(B // TI,),
        in_specs=[
            pl.BlockSpec((B,), lambda i: (0,)),
            pl.BlockSpec((B,), lambda i: (0,)),
        ],
        out_specs=pl.BlockSpec((TI, B), lambda i: (i, 0)),
        out_shape=jax.ShapeDtypeStruct((B, B), jnp.float32),
        scratch_shapes=[pltpu.VMEM((1, B), jnp.float32)],
    )(s, b)


# speedup vs baseline: 1.0720x; 1.0084x over previous
import jax
import jax.numpy as jnp
from jax import lax
from jax.experimental import pallas as pl
from jax.experimental.pallas import tpu as pltpu
from jax.experimental.pallas import tpu_sc as plsc

_L = 16


def _sc_stage(wu_hbm, wv_hbm, ie_hbm, ib_hbm, oe_hbm, ob_hbm,
              s_hbm, b_hbm,
              wu_v, wv_v, urows_v, vrows_v, ib_v, ob_v, tmp_v,
              s_loc, b_loc, sem_u, sem_v, sem_b):
    nc = 2
    pw = wu_v.shape[0]
    wid = lax.axis_index("s") * nc + lax.axis_index("c")
    base = wid * pw

    pltpu.sync_copy(wu_hbm.at[pl.ds(base, pw)], wu_v)
    pltpu.sync_copy(wv_hbm.at[pl.ds(base, pw)], wv_v)
    cp_u = pltpu.async_copy(ie_hbm.at[wu_v], urows_v, sem_u)
    cp_v = pltpu.async_copy(oe_hbm.at[wv_v], vrows_v, sem_v)
    cp_ib = pltpu.async_copy(ib_hbm, ib_v, sem_b)
    cp_ob = pltpu.async_copy(ob_hbm, ob_v, sem_b)

    iota = lax.iota(jnp.int32, _L)
    d = urows_v.shape[1]
    nchunk = d // _L

    cp_ib.wait()
    cp_ob.wait()

    def bias_body(g, carry):
        g0 = g * _L
        wu16 = wu_v[pl.ds(g0, _L)]
        wv16 = wv_v[pl.ds(g0, _L)]
        b_loc[pl.ds(g0, _L)] = (plsc.load_gather(ib_v, [wu16])
                                + plsc.load_gather(ob_v, [wv16]))
        return carry

    lax.fori_loop(0, pw // _L, bias_body, 0)
    cp_u.wait()
    cp_v.wait()

    def group_body(g, carry):
        g0 = g * _L
        for p in range(_L):
            r = g0 + p
            acc = urows_v[r, pl.ds(0, _L)] * vrows_v[r, pl.ds(0, _L)]
            for c in range(1, nchunk):
                acc = acc + (urows_v[r, pl.ds(c * _L, _L)]
                             * vrows_v[r, pl.ds(c * _L, _L)])
            tmp_v[p, pl.ds(0, _L)] = acc
        s16 = plsc.load_gather(tmp_v, [iota, jnp.zeros((_L,), jnp.int32)])
        for col in range(1, _L):
            s16 = s16 + plsc.load_gather(
                tmp_v, [iota, jnp.full((_L,), col, jnp.int32)])
        s_loc[pl.ds(g0, _L)] = s16
        return carry

    lax.fori_loop(0, pw // _L, group_body, 0)

    pltpu.sync_copy(s_loc, s_hbm.at[pl.ds(base, pw)])
    pltpu.sync_copy(b_loc, b_hbm.at[pl.ds(base, pw)])


def _make_sc_stage(B, V, D):
    nw = 32
    pw = B // nw
    mesh = plsc.VectorSubcoreMesh(core_axis_name="c", subcore_axis_name="s")
    return pl.kernel(
        _sc_stage,
        out_type=(jax.ShapeDtypeStruct((B,), jnp.float32),
                  jax.ShapeDtypeStruct((B,), jnp.float32)),
        mesh=mesh,
        compiler_params=pltpu.CompilerParams(
            needs_layout_passes=False,
            disable_bounds_checks=True,
            disable_semaphore_checks=True,
        ),
        scratch_types=[
            pltpu.VMEM((pw,), jnp.int32),
            pltpu.VMEM((pw,), jnp.int32),
            pltpu.VMEM((pw, D), jnp.float32),
            pltpu.VMEM((pw, D), jnp.float32),
            pltpu.VMEM((V,), jnp.float32),
            pltpu.VMEM((V,), jnp.float32),
            pltpu.VMEM((_L, _L), jnp.float32),
            pltpu.VMEM((pw,), jnp.float32),
            pltpu.VMEM((pw,), jnp.float32),
            pltpu.SemaphoreType.DMA,
            pltpu.SemaphoreType.DMA,
            pltpu.SemaphoreType.DMA,
        ],
    )


def _bcast_kernel(s_ref, b_ref, o_ref, srow_ref):
    i = pl.program_id(0)
    TI, B = o_ref.shape

    @pl.when(i == 0)
    def _():
        srow_ref[...] = s_ref[...].reshape(1, B)

    ident = (lax.broadcasted_iota(jnp.int32, (TI, TI), 0)
             == lax.broadcasted_iota(jnp.int32, (TI, TI), 1)
             ).astype(jnp.float32)
    brow = b_ref[pl.ds(i * TI, TI)].reshape(1, TI)
    bcol = lax.dot_general(ident, brow, (((1,), (1,)), ((), ())),
                           preferred_element_type=jnp.float32)
    o_ref[...] = bcol + srow_ref[...]


def kernel(word_u, word_v, in_embed_w, in_bias_w, out_embed_w, out_bias_w):
    B = word_u.shape[0]
    V, D = in_embed_w.shape
    wu = word_u.astype(jnp.int32)
    wv = word_v.astype(jnp.int32)

    s, b = _make_sc_stage(B, V, D)(
        wu, wv, in_embed_w, in_bias_w.reshape(V),
        out_embed_w, out_bias_w.reshape(V))

    TI = 256
    return pl.pallas_call(
        _bcast_kernel,
        grid=(B // TI,),
        in_specs=[
            pl.BlockSpec((B,), lambda i: (0,)),
            pl.BlockSpec((B,), lambda i: (0,)),
        ],
        out_specs=pl.BlockSpec((TI, B), lambda i: (i, 0)),
        out_shape=jax.ShapeDtypeStruct((B, B), jnp.float32),
        scratch_shapes=[pltpu.VMEM((1, B), jnp.float32)],
    )(s, b)
